# baseline (speedup 1.0000x reference)
import functools

import jax
import jax.numpy as jnp
from jax import lax
from jax.experimental import pallas as pl
from jax.experimental.pallas import tpu as pltpu
from jax.experimental.pallas import tpu_sc as plsc

B, S = 4, 2048
NTOK = B * S
DOUT = 192
NSEG = DOUT // 16
NSEG_L = 8
NJ = 7
EPS = 1e-12

NC, NS = 2, 16
NW = NC * NS
SPAN = NTOK // NW
CH = 64
NCH = SPAN // CH


def _tc_table_body(x_ref, y_ref, h_ref, w_ref, bp_ref, w_mat_ref, b_ref,
                   tl_ref, tr_ref):
    f32 = jnp.float32
    embs = (x_ref, y_ref, x_ref, y_ref, h_ref, w_ref)
    for i, e in enumerate(embs):
        d = jnp.dot(e[...], w_mat_ref[i * 128:(i + 1) * 128, :],
                    preferred_element_type=f32)
        tl_ref[i * 1024:(i + 1) * 1024, :] = d[:, 0:128]
        tr_ref[i * 1024:(i + 1) * 1024, 0:64] = d[:, 128:192]
    bp = bp_ref[...] + b_ref[...]
    tl_ref[6144:8192, :] = bp[:, 0:128]
    tr_ref[6144:8192, 0:64] = bp[:, 128:192]
    tr_ref[:, 64:128] = jnp.zeros((NTOK, 64), f32)


def _build_tables(x_emb, y_emb, h_emb, w_emb, box_pos_emb, w_mat, b):
    return pl.pallas_call(
        _tc_table_body,
        out_shape=(jax.ShapeDtypeStruct((NTOK, 128), jnp.float32),
                   jax.ShapeDtypeStruct((NTOK, 128), jnp.float32)),
    )(x_emb, y_emb, h_emb, w_emb, box_pos_emb, w_mat, b.reshape(1, DOUT))


def _sc_body(tl_hbm, tr_hbm, bbox_hbm, pos_hbm, gam_hbm, bet_hbm,
             outl_hbm, outr_hbm,
             bb0, bb1, pp0, pp1, ix0, ix1, al0, al1, ar0, ar1,
             obl_v, obr_v, gam_v, bet_v, sem0, sem1):
    bbs, pps, ixs = (bb0, bb1), (pp0, pp1), (ix0, ix1)
    als, ars, sems = (al0, al1), (ar0, ar1), (sem0, sem1)
    wid = lax.axis_index("s") * NC + lax.axis_index("c")
    pltpu.sync_copy(gam_hbm, gam_v)
    pltpu.sync_copy(bet_hbm, bet_v)

    def fire(ch, bf):
        base = wid * SPAN + ch * CH
        bb, pp, ix = bbs[bf], pps[bf], ixs[bf]
        al, ar = als[bf], ars[bf]
        pltpu.sync_copy(bbox_hbm.at[pl.ds(base * 4, CH * 4)], bb)
        pltpu.sync_copy(pos_hbm.at[pl.ds(base, CH)], pp)
        for g in range(CH // 16):
            row4 = (lax.iota(jnp.int32, 16) + g * 16) * 4
            b0 = plsc.load_gather(bb, [row4])
            b1 = plsc.load_gather(bb, [row4 + 1])
            b2 = plsc.load_gather(bb, [row4 + 2])
            b3 = plsc.load_gather(bb, [row4 + 3])
            p = pp[pl.ds(g * 16, 16)]
            ix[pl.ds(0 * CH + g * 16, 16)] = b0
            ix[pl.ds(1 * CH + g * 16, 16)] = b1 + 1024
            ix[pl.ds(2 * CH + g * 16, 16)] = b2 + 2048
            ix[pl.ds(3 * CH + g * 16, 16)] = b3 + 3072
            ix[pl.ds(4 * CH + g * 16, 16)] = (b3 - b1) + 4096
            ix[pl.ds(5 * CH + g * 16, 16)] = (b2 - b0) + 5120
            ix[pl.ds(6 * CH + g * 16, 16)] = p + 6144
        zero = jnp.zeros((16,), jnp.float32)

        def zero_body(t, c2):
            for c in range(NSEG_L):
                al[t, pl.ds(c * 16, 16)] = zero
            for c in range(NSEG - NSEG_L):
                ar[t, pl.ds(c * 16, 16)] = zero
            return c2
        lax.fori_loop(0, CH, zero_body, 0)
        hs = []
        for j in range(NJ):
            isl = ix.at[pl.ds(j * CH, CH)]
            hs.append(pltpu.async_copy(tl_hbm.at[isl], al, sems[bf], add=True))
            hs.append(pltpu.async_copy(tr_hbm.at[isl], ar, sems[bf], add=True))
        return hs

    def compute(ch, bf):
        base = wid * SPAN + ch * CH
        al, ar = als[bf], ars[bf]

        def tok_body(t, c2):
            s = jnp.zeros((16,), jnp.float32)
            ss = jnp.zeros((16,), jnp.float32)
            for c in range(NSEG):
                if c < NSEG_L:
                    a = al[t, pl.ds(c * 16, 16)]
                else:
                    a = ar[t, pl.ds((c - NSEG_L) * 16, 16)]
                s = s + a
                ss = ss + a * a
            mu_v = jnp.full((16,), jnp.sum(s), jnp.float32) * (1.0 / DOUT)
            var_v = jnp.full((16,), jnp.sum(ss), jnp.float32) * (1.0 / DOUT) - mu_v * mu_v
            v = var_v + EPS
            yi = jnp.full((16,), 0x5F3759DF, jnp.int32) - lax.shift_right_logical(
                plsc.bitcast(v, jnp.int32), jnp.full((16,), 1, jnp.int32))
            r = plsc.bitcast(yi, jnp.float32)
            for _ in range(3):
                r = r * (1.5 - 0.5 * v * r * r)
            for c in range(NSEG):
                csl = pl.ds(c * 16, 16)
                if c < NSEG_L:
                    a = al[t, csl]
                    obl_v[t, csl] = (a - mu_v) * r * gam_v[csl] + bet_v[csl]
                else:
                    rsl = pl.ds((c - NSEG_L) * 16, 16)
                    a = ar[t, rsl]
                    obr_v[t, rsl] = (a - mu_v) * r * gam_v[csl] + bet_v[csl]
            return c2
        lax.fori_loop(0, CH, tok_body, 0)
        pltpu.sync_copy(obl_v, outl_hbm.at[pl.ds(base, CH), :])
        pltpu.sync_copy(obr_v, outr_hbm.at[pl.ds(base, CH), :])

    handles = {0: fire(0, 0), 1: fire(1, 1)}
    for ch in range(NCH):
        bf = ch % 2
        for h in handles.pop(ch):
            h.wait()
        compute(ch, bf)
        if ch + 2 < NCH:
            handles[ch + 2] = fire(ch + 2, bf)


@functools.partial(
    pl.kernel,
    out_type=(jax.ShapeDtypeStruct((NTOK, 128), jnp.float32),
              jax.ShapeDtypeStruct((NTOK, 128), jnp.float32)),
    mesh=plsc.VectorSubcoreMesh(core_axis_name="c", subcore_axis_name="s",
                                num_cores=NC, num_subcores=NS),
    compiler_params=pltpu.CompilerParams(needs_layout_passes=False,
                                         use_tc_tiling_on_sc=True),
    scratch_types=[
        pltpu.VMEM((CH * 4,), jnp.int32),
        pltpu.VMEM((CH * 4,), jnp.int32),
        pltpu.VMEM((CH,), jnp.int32),
        pltpu.VMEM((CH,), jnp.int32),
        pltpu.VMEM((NJ * CH,), jnp.int32),
        pltpu.VMEM((NJ * CH,), jnp.int32),
        pltpu.VMEM((CH, 128), jnp.float32),
        pltpu.VMEM((CH, 128), jnp.float32),
        pltpu.VMEM((CH, 128), jnp.float32),
        pltpu.VMEM((CH, 128), jnp.float32),
        pltpu.VMEM((CH, 128), jnp.float32),
        pltpu.VMEM((CH, 128), jnp.float32),
        pltpu.VMEM((DOUT,), jnp.float32),
        pltpu.VMEM((DOUT,), jnp.float32),
        pltpu.SemaphoreType.DMA,
        pltpu.SemaphoreType.DMA,
    ],
)
def _sc_gather_ln(tl_hbm, tr_hbm, bbox_hbm, pos_hbm, gam_hbm, bet_hbm,
                  outl_hbm, outr_hbm, *rest):
    _sc_body(tl_hbm, tr_hbm, bbox_hbm, pos_hbm, gam_hbm, bet_hbm,
             outl_hbm, outr_hbm, *rest)


def kernel(bbox, position_ids, x_emb, y_emb, h_emb, w_emb, box_pos_emb, W, b, gamma, beta):
    tl, tr = _build_tables(x_emb, y_emb, h_emb, w_emb, box_pos_emb, W, b)
    bbox_flat = bbox.reshape(NTOK * 4).astype(jnp.int32)
    pos_flat = position_ids.reshape(NTOK).astype(jnp.int32)
    outl, outr = _sc_gather_ln(tl, tr, bbox_flat, pos_flat, gamma, beta)
    out = jnp.concatenate([outl, outr[:, :64]], axis=-1)
    return out.reshape(B, S, DOUT)
